# Initial kernel scaffold; baseline (speedup 1.0000x reference)
import jax
import jax.numpy as jnp
from jax.experimental import pallas as pl


def kernel(node_attr, edge_attr, edge_index, coords, frame, W1, b1, W2, b2, Wet, bet, Wnm, bnm, Wem, bem):
    raise NotImplementedError("write your pallas kernel here")


# speedup vs baseline: 4.5034x; 4.5034x over previous
import functools

import jax
import jax.numpy as jnp
from jax import lax
from jax.experimental import pallas as pl
from jax.experimental.pallas import tpu as pltpu
from jax.experimental.pallas import tpu_sc as plsc

N = 10000
E = 50000
NPAD = 10240
EPAD = 51200
K = 64
NC, NS = 2, 16
F1 = 512
F2 = 128
NCH = F1 // F2
RPT = NPAD // NS

@functools.cache
def _mesh():
    return plsc.VectorSubcoreMesh(core_axis_name="c", subcore_axis_name="s")


def _sc_degree(ei1s, ones_slab, z128):
    nrows = (EPAD // K) // (NC * NS)

    @functools.partial(
        pl.kernel, mesh=_mesh(),
        out_type=jax.ShapeDtypeStruct((NC, NPAD, F2), jnp.float32),
        scratch_types=[
            pltpu.VMEM((nrows, K), jnp.int32),
            pltpu.VMEM((K, F2), jnp.float32),
            pltpu.VMEM_SHARED((NPAD + 8, F2), jnp.float32),
        ],
    )
    def k(ei1s_hbm, ones_hbm, z16_hbm, out_hbm, sidx_v, ones_v, acc):
        cid = lax.axis_index("c")
        sid = lax.axis_index("s")
        pltpu.sync_copy(z16_hbm, acc.at[pl.ds(sid * RPT, RPT)])
        pltpu.sync_copy(ones_hbm, ones_v)
        pltpu.sync_copy(ei1s_hbm.at[cid * NS + sid], sidx_v)
        plsc.subcore_barrier()
        for b in range(nrows):
            pltpu.sync_copy(ones_v, acc.at[sidx_v.at[b]], add=True)
        plsc.subcore_barrier()
        pltpu.sync_copy(acc.at[pl.ds(sid * RPT, RPT)],
                        out_hbm.at[cid, pl.ds(sid * RPT, RPT)])

    return k(ei1s, ones_slab, z128)


def _sc_gather_pairsum(i0, i1, ta, tb, width, out_rows):
    nrows = (out_rows // K) // (NC * NS)

    @functools.partial(
        pl.kernel, mesh=_mesh(),
        out_type=jax.ShapeDtypeStruct((out_rows, width), jnp.float32),
        scratch_types=[
            pltpu.VMEM((nrows, K), jnp.int32),
            pltpu.VMEM((nrows, K), jnp.int32),
            pltpu.VMEM((K, width), jnp.float32),
            pltpu.VMEM((K, width), jnp.float32),
            pltpu.SemaphoreType.DMA,
        ],
    )
    def k(i0_hbm, i1_hbm, ta_hbm, tb_hbm, out_hbm, i0_v, i1_v, ra, rb, sem):
        cid = lax.axis_index("c")
        sid = lax.axis_index("s")
        wid = cid * NS + sid
        pltpu.sync_copy(i0_hbm.at[wid], i0_v)
        pltpu.sync_copy(i1_hbm.at[wid], i1_v)
        for b in range(nrows):
            pltpu.async_copy(ta_hbm.at[i0_v.at[b]], ra, sem).wait()
            pltpu.async_copy(tb_hbm.at[i1_v.at[b]], rb, sem).wait()

            def add_row(r, carry):
                for c in range(width // 16):
                    ra[r, pl.ds(c * 16, 16)] = (ra[r, pl.ds(c * 16, 16)]
                                                + rb[r, pl.ds(c * 16, 16)])
                return carry

            lax.fori_loop(0, K, add_row, 0)
            pltpu.sync_copy(ra, out_hbm.at[pl.ds(wid * nrows * K + b * K, K)])

    return k(i0, i1, ta, tb)


def _sc_scatter1(hsflat, i0, i1s):
    nrows = (EPAD // K) // NS

    @functools.partial(
        pl.kernel, mesh=_mesh(),
        out_type=jax.ShapeDtypeStruct((NCH, NPAD, F2), jnp.float32),
        scratch_types=[
            pltpu.VMEM((nrows, K), jnp.int32),
            pltpu.VMEM((nrows, K), jnp.int32),
            pltpu.VMEM((K,), jnp.int32),
            pltpu.VMEM((K, F2), jnp.float32),
            pltpu.VMEM_SHARED((NPAD + 8, F2), jnp.float32),
            pltpu.SemaphoreType.DMA,
        ],
    )
    def k(hs_hbm, i0_hbm, i1s_hbm, out_hbm, i0_v, si_v, g1, rows_v, acc, sem):
        cid = lax.axis_index("c")
        sid = lax.axis_index("s")
        pltpu.sync_copy(i0_hbm.at[sid], i0_v)
        pltpu.sync_copy(i1s_hbm.at[sid], si_v)
        for r in range(NCH // NC):
            fc = cid * (NCH // NC) + r
            off = fc * NPAD
            pltpu.sync_copy(hs_hbm.at[pl.ds(off + sid * RPT, RPT)],
                            acc.at[pl.ds(sid * RPT, RPT)])
            plsc.subcore_barrier()
            for b in range(nrows):
                for i in range(K // 16):
                    g1[pl.ds(i * 16, 16)] = i0_v[b, pl.ds(i * 16, 16)] + off
                pltpu.async_copy(hs_hbm.at[g1], rows_v, sem).wait()
                pltpu.sync_copy(rows_v, acc.at[si_v.at[b]], add=True)
            plsc.subcore_barrier()
            pltpu.sync_copy(acc.at[pl.ds(sid * RPT, RPT)],
                            out_hbm.at[fc, pl.ds(sid * RPT, RPT)])

    return k(hsflat, i0, i1s)


def _sc_scatter2(y, i0, i1s, z128):
    nrows = (EPAD // K) // (NC * NS)

    @functools.partial(
        pl.kernel, mesh=_mesh(),
        out_type=jax.ShapeDtypeStruct((NC, NPAD, F2), jnp.float32),
        scratch_types=[
            pltpu.VMEM((nrows, K), jnp.int32),
            pltpu.VMEM((nrows, K), jnp.int32),
            pltpu.VMEM((K, F2), jnp.float32),
            pltpu.VMEM_SHARED((NPAD + 8, F2), jnp.float32),
            pltpu.SemaphoreType.DMA,
        ],
    )
    def k(y_hbm, i0_hbm, i1s_hbm, z128_hbm, out_hbm, i0_v, si_v, rows_v, acc, sem):
        cid = lax.axis_index("c")
        sid = lax.axis_index("s")

        @pl.when(cid == 0)
        def _():
            pltpu.sync_copy(y_hbm.at[pl.ds(sid * RPT, RPT)],
                            acc.at[pl.ds(sid * RPT, RPT)])

        @pl.when(cid != 0)
        def _():
            pltpu.sync_copy(z128_hbm, acc.at[pl.ds(sid * RPT, RPT)])

        wid = cid * NS + sid
        pltpu.sync_copy(i0_hbm.at[wid], i0_v)
        pltpu.sync_copy(i1s_hbm.at[wid], si_v)
        plsc.subcore_barrier()
        for b in range(nrows):
            pltpu.async_copy(y_hbm.at[i0_v.at[b]], rows_v, sem).wait()
            pltpu.sync_copy(rows_v, acc.at[si_v.at[b]], add=True)
        plsc.subcore_barrier()
        pltpu.sync_copy(acc.at[pl.ds(sid * RPT, RPT)],
                        out_hbm.at[cid, pl.ds(sid * RPT, RPT)])

    return k(y, i0, i1s, z128)


def _tc_dense1(node_attr, w1a, w1b):
    bm = 400

    def body(x_ref, wa_ref, wb_ref, a_ref, b_ref):
        x = x_ref[...]
        a_ref[...] = jnp.dot(x, wa_ref[...], preferred_element_type=jnp.float32)
        b_ref[...] = jnp.dot(x, wb_ref[...], preferred_element_type=jnp.float32)

    return pl.pallas_call(
        body,
        grid=(N // bm,),
        in_specs=[
            pl.BlockSpec((bm, 1024), lambda i: (i, 0)),
            pl.BlockSpec((1024, F1), lambda i: (0, 0)),
            pl.BlockSpec((1024, F1), lambda i: (0, 0)),
        ],
        out_specs=[
            pl.BlockSpec((bm, F1), lambda i: (i, 0)),
            pl.BlockSpec((bm, F1), lambda i: (i, 0)),
        ],
        out_shape=[
            jax.ShapeDtypeStruct((N, F1), jnp.float32),
            jax.ShapeDtypeStruct((N, F1), jnp.float32),
        ],
    )(node_attr, w1a, w1b)


def _tc_edgefeat(edge_attr, wet, bet, wem, bem, w2c3):
    be = 400

    def body(ea_ref, wet_ref, bet_ref, wem_ref, bem_ref, w_ref,
             eat_ref, efw_ref):
        ea = ea_ref[...]
        eat = (ea[:, 0:1] * wet_ref[0, 0] + ea[:, 1:2] * wet_ref[1, 0]
               + bet_ref[0, 0])
        eat_ref[...] = eat
        ef = jax.nn.relu(eat * wem_ref[...] + bem_ref[...])
        efw_ref[...] = jnp.dot(ef, w_ref[...], preferred_element_type=jnp.float32)

    return pl.pallas_call(
        body,
        grid=(E // be,),
        in_specs=[
            pl.BlockSpec((be, 2), lambda i: (i, 0)),
            pl.BlockSpec((2, 1), lambda i: (0, 0)),
            pl.BlockSpec((1, 1), lambda i: (0, 0)),
            pl.BlockSpec((1, F2), lambda i: (0, 0)),
            pl.BlockSpec((1, F2), lambda i: (0, 0)),
            pl.BlockSpec((F2, F2), lambda i: (0, 0)),
        ],
        out_specs=[
            pl.BlockSpec((be, 1), lambda i: (i, 0)),
            pl.BlockSpec((be, F2), lambda i: (i, 0)),
        ],
        out_shape=[
            jax.ShapeDtypeStruct((E, 1), jnp.float32),
            jax.ShapeDtypeStruct((E, F2), jnp.float32),
        ],
    )(edge_attr, wet, bet, wem, bem, w2c3)


def _tc_assemble_hs(g, cnt2, eat, w1c):
    bn = 512

    def body(g_ref, cnt_ref, eat_ref, wc_ref, hs_ref):
        cnt = cnt_ref[0, :, 0] + cnt_ref[1, :, 0]
        dinv = lax.rsqrt(1.0 + cnt)[:, None]
        h = dinv * (g_ref[...] + eat_ref[...] * wc_ref[...])
        hs_ref[...] = h.reshape(bn, NCH, F2).transpose(1, 0, 2)

    return pl.pallas_call(
        body,
        grid=(NPAD // bn,),
        in_specs=[
            pl.BlockSpec((bn, F1), lambda i: (i, 0)),
            pl.BlockSpec((2, bn, F2), lambda i: (0, i, 0)),
            pl.BlockSpec((bn, 1), lambda i: (i, 0)),
            pl.BlockSpec((1, F1), lambda i: (0, 0)),
        ],
        out_specs=pl.BlockSpec((NCH, bn, F2), lambda i: (0, i, 0)),
        out_shape=jax.ShapeDtypeStruct((NCH, NPAD, F2), jnp.float32),
    )(g, cnt2, eat, w1c)


def _tc_dense2(z1, cnt2, b1, wnm, bnm, w2a, w2b, w2c1, w2c2):
    bn = 512

    def body(z_ref, cnt_ref, b1_ref, wnm_ref, bnm_ref, wa_ref, wb_ref,
             wc1_ref, wc2_ref, pa_ref, pb_ref):
        z = z_ref[...].transpose(1, 0, 2).reshape(bn, F1)
        cnt = cnt_ref[0, :, 0] + cnt_ref[1, :, 0]
        dinv = lax.rsqrt(1.0 + cnt)[:, None]
        out1 = jax.nn.relu(dinv * z + b1_ref[...])
        q = jax.nn.relu(jnp.dot(out1, wnm_ref[...],
                                preferred_element_type=jnp.float32)
                        + bnm_ref[...])
        pa_ref[...] = (jnp.dot(out1, wa_ref[...], preferred_element_type=jnp.float32)
                       + jnp.dot(q, wc1_ref[...], preferred_element_type=jnp.float32))
        pb_ref[...] = (jnp.dot(out1, wb_ref[...], preferred_element_type=jnp.float32)
                       + jnp.dot(q, wc2_ref[...], preferred_element_type=jnp.float32))

    return pl.pallas_call(
        body,
        grid=(NPAD // bn,),
        in_specs=[
            pl.BlockSpec((NCH, bn, F2), lambda i: (0, i, 0)),
            pl.BlockSpec((2, bn, F2), lambda i: (0, i, 0)),
            pl.BlockSpec((1, F1), lambda i: (0, 0)),
            pl.BlockSpec((F1, F2), lambda i: (0, 0)),
            pl.BlockSpec((1, F2), lambda i: (0, 0)),
            pl.BlockSpec((F1, F2), lambda i: (0, 0)),
            pl.BlockSpec((F1, F2), lambda i: (0, 0)),
            pl.BlockSpec((F2, F2), lambda i: (0, 0)),
            pl.BlockSpec((F2, F2), lambda i: (0, 0)),
        ],
        out_specs=[
            pl.BlockSpec((bn, F2), lambda i: (i, 0)),
            pl.BlockSpec((bn, F2), lambda i: (i, 0)),
        ],
        out_shape=[
            jax.ShapeDtypeStruct((NPAD, F2), jnp.float32),
            jax.ShapeDtypeStruct((NPAD, F2), jnp.float32),
        ],
    )(z1, cnt2, b1, wnm, bnm, w2a, w2b, w2c1, w2c2)


def _tc_combine_y(pg, efw, cnt2):
    be = 400
    nblk = N // be

    def body(pg_ref, efw_ref, cnt_ref, y_ref):
        pid = pl.program_id(0)
        x2 = pg_ref[...] + efw_ref[...]
        cnt = cnt_ref[0, :, 0] + cnt_ref[1, :, 0]
        dinv = lax.rsqrt(1.0 + cnt)[:, None]
        y_ref[...] = jnp.where(pid < nblk, dinv * x2, x2)

    return pl.pallas_call(
        body,
        grid=(E // be,),
        in_specs=[
            pl.BlockSpec((be, F2), lambda i: (i, 0)),
            pl.BlockSpec((be, F2), lambda i: (i, 0)),
            pl.BlockSpec((2, be, F2), lambda i: (0, jnp.minimum(i, nblk - 1), 0)),
        ],
        out_specs=pl.BlockSpec((be, F2), lambda i: (i, 0)),
        out_shape=jax.ShapeDtypeStruct((E, F2), jnp.float32),
    )(pg, efw, cnt2)


def _tc_final(z2, y, cnt2, b2):
    be = 400
    nblk = N // be

    def body(z_ref, y_ref, cnt_ref, b2_ref, out_ref):
        pid = pl.program_id(0)
        cnt = cnt_ref[0, :, 0] + cnt_ref[1, :, 0]
        dinv = lax.rsqrt(1.0 + cnt)[:, None]
        head = dinv * (z_ref[0] + z_ref[1])
        out_ref[...] = jnp.where(pid < nblk, head, y_ref[...]) + b2_ref[...]

    return pl.pallas_call(
        body,
        grid=(E // be,),
        in_specs=[
            pl.BlockSpec((2, be, F2), lambda i: (0, jnp.minimum(i, nblk - 1), 0)),
            pl.BlockSpec((be, F2), lambda i: (i, 0)),
            pl.BlockSpec((2, be, F2), lambda i: (0, jnp.minimum(i, nblk - 1), 0)),
            pl.BlockSpec((1, F2), lambda i: (0, 0)),
        ],
        out_specs=pl.BlockSpec((be, F2), lambda i: (i, 0)),
        out_shape=jax.ShapeDtypeStruct((E, F2), jnp.float32),
    )(z2, y, cnt2, b2)


def kernel(node_attr, edge_attr, edge_index, coords, frame,
           W1, b1, W2, b2, Wet, bet, Wnm, bnm, Wem, bem):
    del coords, frame
    f32 = jnp.float32
    ei0 = edge_index[0].astype(jnp.int32)
    ei1 = edge_index[1].astype(jnp.int32)

    ei0g = jnp.pad(ei0, (0, EPAD - E)).reshape(EPAD // K, K)
    ei1g = jnp.pad(ei1, (0, EPAD - E)).reshape(EPAD // K, K)
    ei1s = jnp.pad(ei1, (0, EPAD - E),
                   constant_values=NPAD).reshape(EPAD // K, K)

    ones_slab = jnp.ones((K, F2), f32)
    z128 = jnp.zeros((RPT, F2), f32)

    w1a = W1[:1024]
    w1b = W1[1024:2048]
    w1c = W1[2048:2049]
    w2a, w2b = W2[:F1], W2[F1:2 * F1]
    w2c1 = W2[2 * F1:2 * F1 + F2]
    w2c2 = W2[2 * F1 + F2:2 * F1 + 2 * F2]
    w2c3 = W2[2 * F1 + 2 * F2:]
    b1r = b1.reshape(1, F1)
    b2r = b2.reshape(1, F2)
    bnmr = bnm.reshape(1, F2)
    wemr = Wem.reshape(1, F2)
    bemr = bem.reshape(1, F2)
    betr = bet.reshape(1, 1)

    nre = (EPAD // K) // (NC * NS)
    ei0g_w = ei0g.reshape(NC * NS, nre, K)
    ei1g_w = ei1g.reshape(NC * NS, nre, K)
    ei1s_w = ei1s.reshape(NC * NS, nre, K)
    ei0g_s = ei0g.reshape(NS, nre * NC, K)
    ei1s_s = ei1s.reshape(NS, nre * NC, K)
    nrn = (NPAD // K) // (NC * NS)
    ei0g_n = ei0g[:NPAD // K].reshape(NC * NS, nrn, K)
    ei1g_n = ei1g[:NPAD // K].reshape(NC * NS, nrn, K)

    cnt2 = _sc_degree(ei1s_w, ones_slab, z128)
    a_tab, b_tab = _tc_dense1(node_attr, w1a, w1b)
    eat, efw = _tc_edgefeat(edge_attr, Wet, betr, wemr, bemr, w2c3)
    g = _sc_gather_pairsum(ei0g_n, ei1g_n, a_tab, b_tab, F1, NPAD)
    hs = _tc_assemble_hs(g, cnt2, eat[:NPAD], w1c)
    z1 = _sc_scatter1(hs.reshape(NCH * NPAD, F2), ei0g_s, ei1s_s)
    pa, pb = _tc_dense2(z1, cnt2, b1r, Wnm, bnmr, w2a, w2b, w2c1, w2c2)
    pg = _sc_gather_pairsum(ei0g_w, ei1g_w, pa, pb, F2, EPAD)
    y = _tc_combine_y(pg[:E], efw, cnt2)
    z2 = _sc_scatter2(y, ei0g_w, ei1s_w, z128)
    return _tc_final(z2, y, cnt2, b2r)


# speedup vs baseline: 5.1387x; 1.1411x over previous
import functools

import jax
import jax.numpy as jnp
from jax import lax
from jax.experimental import pallas as pl
from jax.experimental.pallas import tpu as pltpu
from jax.experimental.pallas import tpu_sc as plsc

N = 10000
E = 50000
NPAD = 10240
EPAD = 51200
K = 64
NC, NS = 2, 16
F1 = 512
F2 = 128
NCH = F1 // F2
RPT = NPAD // NS

@functools.cache
def _mesh():
    return plsc.VectorSubcoreMesh(core_axis_name="c", subcore_axis_name="s")


def _sc_degree(ei1s, ones_slab, z128):
    nrows = (EPAD // K) // (NC * NS)

    @functools.partial(
        pl.kernel, mesh=_mesh(),
        out_type=jax.ShapeDtypeStruct((NC, NPAD, F2), jnp.float32),
        scratch_types=[
            pltpu.VMEM((nrows, K), jnp.int32),
            pltpu.VMEM((K, F2), jnp.float32),
            pltpu.VMEM_SHARED((NPAD + 8, F2), jnp.float32),
        ],
    )
    def k(ei1s_hbm, ones_hbm, z16_hbm, out_hbm, sidx_v, ones_v, acc):
        cid = lax.axis_index("c")
        sid = lax.axis_index("s")
        pltpu.sync_copy(z16_hbm, acc.at[pl.ds(sid * RPT, RPT)])
        pltpu.sync_copy(ones_hbm, ones_v)
        pltpu.sync_copy(ei1s_hbm.at[cid * NS + sid], sidx_v)
        plsc.subcore_barrier()
        for b in range(nrows):
            pltpu.sync_copy(ones_v, acc.at[sidx_v.at[b]], add=True)
        plsc.subcore_barrier()
        pltpu.sync_copy(acc.at[pl.ds(sid * RPT, RPT)],
                        out_hbm.at[cid, pl.ds(sid * RPT, RPT)])

    return k(ei1s, ones_slab, z128)


def _sc_gather_pairsum(i0, i1, ta, tb, width, out_rows):
    nrows = (out_rows // K) // (NC * NS)

    @functools.partial(
        pl.kernel, mesh=_mesh(),
        out_type=jax.ShapeDtypeStruct((out_rows, width), jnp.float32),
        scratch_types=[
            pltpu.VMEM((nrows, K), jnp.int32),
            pltpu.VMEM((nrows, K), jnp.int32),
            pltpu.VMEM((K, width), jnp.float32),
            pltpu.VMEM((K, width), jnp.float32),
            pltpu.SemaphoreType.DMA,
            pltpu.SemaphoreType.DMA,
            pltpu.SemaphoreType.DMA,
        ],
    )
    def k(i0_hbm, i1_hbm, ta_hbm, tb_hbm, out_hbm, i0_v, i1_v, r0, r1,
          sem_a, sem_b, sem_w):
        cid = lax.axis_index("c")
        sid = lax.axis_index("s")
        wid = cid * NS + sid
        pltpu.sync_copy(i0_hbm.at[wid], i0_v)
        pltpu.sync_copy(i1_hbm.at[wid], i1_v)
        if width <= 128:
            bufs = [r0, r1]
            wout = [None, None]
            pltpu.async_copy(ta_hbm.at[i0_v.at[0]], r0, sem_a).wait()
            badd = pltpu.async_copy(tb_hbm.at[i1_v.at[0]], r0, sem_b, add=True)
            for b in range(nrows):
                cur, nxt = bufs[b % 2], bufs[(b + 1) % 2]
                a_nxt = None
                if b + 1 < nrows:
                    if wout[(b + 1) % 2] is not None:
                        wout[(b + 1) % 2].wait()
                        wout[(b + 1) % 2] = None
                    a_nxt = pltpu.async_copy(ta_hbm.at[i0_v.at[b + 1]], nxt,
                                             sem_a)
                badd.wait()
                wout[b % 2] = pltpu.async_copy(
                    cur, out_hbm.at[pl.ds(wid * nrows * K + b * K, K)], sem_w)
                if a_nxt is not None:
                    a_nxt.wait()
                    badd = pltpu.async_copy(tb_hbm.at[i1_v.at[b + 1]], nxt,
                                            sem_b, add=True)
            for d in wout:
                if d is not None:
                    d.wait()
        else:
            for b in range(nrows):
                da = pltpu.async_copy(ta_hbm.at[i0_v.at[b]], r0, sem_a)
                db = pltpu.async_copy(tb_hbm.at[i1_v.at[b]], r1, sem_b)
                da.wait()
                db.wait()

                def add_row(r2, carry):
                    for c in range(width // 16):
                        r0[r2, pl.ds(c * 16, 16)] = (r0[r2, pl.ds(c * 16, 16)]
                                                     + r1[r2, pl.ds(c * 16, 16)])
                    return carry

                lax.fori_loop(0, K, add_row, 0)
                pltpu.sync_copy(r0, out_hbm.at[pl.ds(wid * nrows * K + b * K, K)])

    return k(i0, i1, ta, tb)


def _sc_scatter1(hsflat, i0, i1s):
    nrows = (EPAD // K) // NS

    @functools.partial(
        pl.kernel, mesh=_mesh(),
        out_type=jax.ShapeDtypeStruct((NCH, NPAD, F2), jnp.float32),
        scratch_types=[
            pltpu.VMEM((nrows, K), jnp.int32),
            pltpu.VMEM((nrows, K), jnp.int32),
            pltpu.VMEM((nrows, K), jnp.int32),
            pltpu.VMEM((K, F2), jnp.float32),
            pltpu.VMEM((K, F2), jnp.float32),
            pltpu.VMEM_SHARED((NPAD + 8, F2), jnp.float32),
            pltpu.SemaphoreType.DMA,
        ],
    )
    def k(hs_hbm, i0_hbm, i1s_hbm, out_hbm, i0_v, si_v, gi_v, rows_v,
          rows_v2, acc, sem):
        cid = lax.axis_index("c")
        sid = lax.axis_index("s")
        pltpu.sync_copy(i0_hbm.at[sid], i0_v)
        pltpu.sync_copy(i1s_hbm.at[sid], si_v)
        for r in range(NCH // NC):
            fc = cid * (NCH // NC) + r
            off = fc * NPAD
            pltpu.sync_copy(hs_hbm.at[pl.ds(off + sid * RPT, RPT)],
                            acc.at[pl.ds(sid * RPT, RPT)])

            def mkidx(b, carry):
                for i in range(K // 16):
                    gi_v[b, pl.ds(i * 16, 16)] = (i0_v[b, pl.ds(i * 16, 16)]
                                                  + off)
                return carry

            lax.fori_loop(0, nrows, mkidx, 0)
            plsc.subcore_barrier()
            bufs = [rows_v, rows_v2]
            g = pltpu.async_copy(hs_hbm.at[gi_v.at[0]], rows_v, sem)
            for b in range(nrows):
                g.wait()
                if b + 1 < nrows:
                    g = pltpu.async_copy(hs_hbm.at[gi_v.at[b + 1]],
                                         bufs[(b + 1) % 2], sem)
                pltpu.sync_copy(bufs[b % 2], acc.at[si_v.at[b]], add=True)
            plsc.subcore_barrier()
            pltpu.sync_copy(acc.at[pl.ds(sid * RPT, RPT)],
                            out_hbm.at[fc, pl.ds(sid * RPT, RPT)])

    return k(hsflat, i0, i1s)


def _sc_scatter2(y, i0, i1s, z128):
    nrows = (EPAD // K) // (NC * NS)

    @functools.partial(
        pl.kernel, mesh=_mesh(),
        out_type=jax.ShapeDtypeStruct((NC, NPAD, F2), jnp.float32),
        scratch_types=[
            pltpu.VMEM((nrows, K), jnp.int32),
            pltpu.VMEM((nrows, K), jnp.int32),
            pltpu.VMEM((K, F2), jnp.float32),
            pltpu.VMEM((K, F2), jnp.float32),
            pltpu.VMEM_SHARED((NPAD + 8, F2), jnp.float32),
            pltpu.SemaphoreType.DMA,
        ],
    )
    def k(y_hbm, i0_hbm, i1s_hbm, z128_hbm, out_hbm, i0_v, si_v, rows_v,
          rows_v2, acc, sem):
        cid = lax.axis_index("c")
        sid = lax.axis_index("s")

        @pl.when(cid == 0)
        def _():
            pltpu.sync_copy(y_hbm.at[pl.ds(sid * RPT, RPT)],
                            acc.at[pl.ds(sid * RPT, RPT)])

        @pl.when(cid != 0)
        def _():
            pltpu.sync_copy(z128_hbm, acc.at[pl.ds(sid * RPT, RPT)])

        wid = cid * NS + sid
        pltpu.sync_copy(i0_hbm.at[wid], i0_v)
        pltpu.sync_copy(i1s_hbm.at[wid], si_v)
        plsc.subcore_barrier()
        bufs = [rows_v, rows_v2]
        g = pltpu.async_copy(y_hbm.at[i0_v.at[0]], rows_v, sem)
        for b in range(nrows):
            g.wait()
            if b + 1 < nrows:
                g = pltpu.async_copy(y_hbm.at[i0_v.at[b + 1]],
                                     bufs[(b + 1) % 2], sem)
            pltpu.sync_copy(bufs[b % 2], acc.at[si_v.at[b]], add=True)
        plsc.subcore_barrier()
        pltpu.sync_copy(acc.at[pl.ds(sid * RPT, RPT)],
                        out_hbm.at[cid, pl.ds(sid * RPT, RPT)])

    return k(y, i0, i1s, z128)


def _tc_dense1(node_attr, w1a, w1b):
    bm = 400

    def body(x_ref, wa_ref, wb_ref, a_ref, b_ref):
        x = x_ref[...]
        a_ref[...] = jnp.dot(x, wa_ref[...], preferred_element_type=jnp.float32)
        b_ref[...] = jnp.dot(x, wb_ref[...], preferred_element_type=jnp.float32)

    return pl.pallas_call(
        body,
        grid=(N // bm,),
        in_specs=[
            pl.BlockSpec((bm, 1024), lambda i: (i, 0)),
            pl.BlockSpec((1024, F1), lambda i: (0, 0)),
            pl.BlockSpec((1024, F1), lambda i: (0, 0)),
        ],
        out_specs=[
            pl.BlockSpec((bm, F1), lambda i: (i, 0)),
            pl.BlockSpec((bm, F1), lambda i: (i, 0)),
        ],
        out_shape=[
            jax.ShapeDtypeStruct((N, F1), jnp.float32),
            jax.ShapeDtypeStruct((N, F1), jnp.float32),
        ],
    )(node_attr, w1a, w1b)


def _tc_edgefeat(edge_attr, wet, bet, wem, bem, w2c3):
    be = 400

    def body(ea_ref, wet_ref, bet_ref, wem_ref, bem_ref, w_ref,
             eat_ref, efw_ref):
        ea = ea_ref[...]
        eat = (ea[:, 0:1] * wet_ref[0, 0] + ea[:, 1:2] * wet_ref[1, 0]
               + bet_ref[0, 0])
        eat_ref[...] = eat
        ef = jax.nn.relu(eat * wem_ref[...] + bem_ref[...])
        efw_ref[...] = jnp.dot(ef, w_ref[...], preferred_element_type=jnp.float32)

    return pl.pallas_call(
        body,
        grid=(E // be,),
        in_specs=[
            pl.BlockSpec((be, 2), lambda i: (i, 0)),
            pl.BlockSpec((2, 1), lambda i: (0, 0)),
            pl.BlockSpec((1, 1), lambda i: (0, 0)),
            pl.BlockSpec((1, F2), lambda i: (0, 0)),
            pl.BlockSpec((1, F2), lambda i: (0, 0)),
            pl.BlockSpec((F2, F2), lambda i: (0, 0)),
        ],
        out_specs=[
            pl.BlockSpec((be, 1), lambda i: (i, 0)),
            pl.BlockSpec((be, F2), lambda i: (i, 0)),
        ],
        out_shape=[
            jax.ShapeDtypeStruct((E, 1), jnp.float32),
            jax.ShapeDtypeStruct((E, F2), jnp.float32),
        ],
    )(edge_attr, wet, bet, wem, bem, w2c3)


def _tc_assemble_hs(g, cnt2, eat, w1c):
    bn = 512

    def body(g_ref, cnt_ref, eat_ref, wc_ref, hs_ref):
        cnt = cnt_ref[0, :, 0] + cnt_ref[1, :, 0]
        dinv = lax.rsqrt(1.0 + cnt)[:, None]
        h = dinv * (g_ref[...] + eat_ref[...] * wc_ref[...])
        hs_ref[...] = h.reshape(bn, NCH, F2).transpose(1, 0, 2)

    return pl.pallas_call(
        body,
        grid=(NPAD // bn,),
        in_specs=[
            pl.BlockSpec((bn, F1), lambda i: (i, 0)),
            pl.BlockSpec((2, bn, F2), lambda i: (0, i, 0)),
            pl.BlockSpec((bn, 1), lambda i: (i, 0)),
            pl.BlockSpec((1, F1), lambda i: (0, 0)),
        ],
        out_specs=pl.BlockSpec((NCH, bn, F2), lambda i: (0, i, 0)),
        out_shape=jax.ShapeDtypeStruct((NCH, NPAD, F2), jnp.float32),
    )(g, cnt2, eat, w1c)


def _tc_dense2(z1, cnt2, b1, wnm, bnm, w2a, w2b, w2c1, w2c2):
    bn = 512

    def body(z_ref, cnt_ref, b1_ref, wnm_ref, bnm_ref, wa_ref, wb_ref,
             wc1_ref, wc2_ref, pa_ref, pb_ref):
        z = z_ref[...].transpose(1, 0, 2).reshape(bn, F1)
        cnt = cnt_ref[0, :, 0] + cnt_ref[1, :, 0]
        dinv = lax.rsqrt(1.0 + cnt)[:, None]
        out1 = jax.nn.relu(dinv * z + b1_ref[...])
        q = jax.nn.relu(jnp.dot(out1, wnm_ref[...],
                                preferred_element_type=jnp.float32)
                        + bnm_ref[...])
        pa_ref[...] = (jnp.dot(out1, wa_ref[...], preferred_element_type=jnp.float32)
                       + jnp.dot(q, wc1_ref[...], preferred_element_type=jnp.float32))
        pb_ref[...] = (jnp.dot(out1, wb_ref[...], preferred_element_type=jnp.float32)
                       + jnp.dot(q, wc2_ref[...], preferred_element_type=jnp.float32))

    return pl.pallas_call(
        body,
        grid=(NPAD // bn,),
        in_specs=[
            pl.BlockSpec((NCH, bn, F2), lambda i: (0, i, 0)),
            pl.BlockSpec((2, bn, F2), lambda i: (0, i, 0)),
            pl.BlockSpec((1, F1), lambda i: (0, 0)),
            pl.BlockSpec((F1, F2), lambda i: (0, 0)),
            pl.BlockSpec((1, F2), lambda i: (0, 0)),
            pl.BlockSpec((F1, F2), lambda i: (0, 0)),
            pl.BlockSpec((F1, F2), lambda i: (0, 0)),
            pl.BlockSpec((F2, F2), lambda i: (0, 0)),
            pl.BlockSpec((F2, F2), lambda i: (0, 0)),
        ],
        out_specs=[
            pl.BlockSpec((bn, F2), lambda i: (i, 0)),
            pl.BlockSpec((bn, F2), lambda i: (i, 0)),
        ],
        out_shape=[
            jax.ShapeDtypeStruct((NPAD, F2), jnp.float32),
            jax.ShapeDtypeStruct((NPAD, F2), jnp.float32),
        ],
    )(z1, cnt2, b1, wnm, bnm, w2a, w2b, w2c1, w2c2)


def _tc_combine_y(pg, efw, cnt2):
    be = 400
    nblk = N // be

    def body(pg_ref, efw_ref, cnt_ref, y_ref):
        pid = pl.program_id(0)
        x2 = pg_ref[...] + efw_ref[...]
        cnt = cnt_ref[0, :, 0] + cnt_ref[1, :, 0]
        dinv = lax.rsqrt(1.0 + cnt)[:, None]
        y_ref[...] = jnp.where(pid < nblk, dinv * x2, x2)

    return pl.pallas_call(
        body,
        grid=(E // be,),
        in_specs=[
            pl.BlockSpec((be, F2), lambda i: (i, 0)),
            pl.BlockSpec((be, F2), lambda i: (i, 0)),
            pl.BlockSpec((2, be, F2), lambda i: (0, jnp.minimum(i, nblk - 1), 0)),
        ],
        out_specs=pl.BlockSpec((be, F2), lambda i: (i, 0)),
        out_shape=jax.ShapeDtypeStruct((E, F2), jnp.float32),
    )(pg, efw, cnt2)


def _tc_final(z2, y, cnt2, b2):
    be = 400
    nblk = N // be

    def body(z_ref, y_ref, cnt_ref, b2_ref, out_ref):
        pid = pl.program_id(0)
        cnt = cnt_ref[0, :, 0] + cnt_ref[1, :, 0]
        dinv = lax.rsqrt(1.0 + cnt)[:, None]
        head = dinv * (z_ref[0] + z_ref[1])
        out_ref[...] = jnp.where(pid < nblk, head, y_ref[...]) + b2_ref[...]

    return pl.pallas_call(
        body,
        grid=(E // be,),
        in_specs=[
            pl.BlockSpec((2, be, F2), lambda i: (0, jnp.minimum(i, nblk - 1), 0)),
            pl.BlockSpec((be, F2), lambda i: (i, 0)),
            pl.BlockSpec((2, be, F2), lambda i: (0, jnp.minimum(i, nblk - 1), 0)),
            pl.BlockSpec((1, F2), lambda i: (0, 0)),
        ],
        out_specs=pl.BlockSpec((be, F2), lambda i: (i, 0)),
        out_shape=jax.ShapeDtypeStruct((E, F2), jnp.float32),
    )(z2, y, cnt2, b2)


def kernel(node_attr, edge_attr, edge_index, coords, frame,
           W1, b1, W2, b2, Wet, bet, Wnm, bnm, Wem, bem):
    del coords, frame
    f32 = jnp.float32
    ei0 = edge_index[0].astype(jnp.int32)
    ei1 = edge_index[1].astype(jnp.int32)

    ei0g = jnp.pad(ei0, (0, EPAD - E)).reshape(EPAD // K, K)
    ei1g = jnp.pad(ei1, (0, EPAD - E)).reshape(EPAD // K, K)
    ei1s = jnp.pad(ei1, (0, EPAD - E),
                   constant_values=NPAD).reshape(EPAD // K, K)

    ones_slab = jnp.ones((K, F2), f32)
    z128 = jnp.zeros((RPT, F2), f32)

    w1a = W1[:1024]
    w1b = W1[1024:2048]
    w1c = W1[2048:2049]
    w2a, w2b = W2[:F1], W2[F1:2 * F1]
    w2c1 = W2[2 * F1:2 * F1 + F2]
    w2c2 = W2[2 * F1 + F2:2 * F1 + 2 * F2]
    w2c3 = W2[2 * F1 + 2 * F2:]
    b1r = b1.reshape(1, F1)
    b2r = b2.reshape(1, F2)
    bnmr = bnm.reshape(1, F2)
    wemr = Wem.reshape(1, F2)
    bemr = bem.reshape(1, F2)
    betr = bet.reshape(1, 1)

    nre = (EPAD // K) // (NC * NS)
    ei0g_w = ei0g.reshape(NC * NS, nre, K)
    ei1g_w = ei1g.reshape(NC * NS, nre, K)
    ei1s_w = ei1s.reshape(NC * NS, nre, K)
    ei0g_s = ei0g.reshape(NS, nre * NC, K)
    ei1s_s = ei1s.reshape(NS, nre * NC, K)
    nrn = (NPAD // K) // (NC * NS)
    ei0g_n = ei0g[:NPAD // K].reshape(NC * NS, nrn, K)
    ei1g_n = ei1g[:NPAD // K].reshape(NC * NS, nrn, K)

    cnt2 = _sc_degree(ei1s_w, ones_slab, z128)
    a_tab, b_tab = _tc_dense1(node_attr, w1a, w1b)
    eat, efw = _tc_edgefeat(edge_attr, Wet, betr, wemr, bemr, w2c3)
    g = _sc_gather_pairsum(ei0g_n, ei1g_n, a_tab, b_tab, F1, NPAD)
    hs = _tc_assemble_hs(g, cnt2, eat[:NPAD], w1c)
    z1 = _sc_scatter1(hs.reshape(NCH * NPAD, F2), ei0g_s, ei1s_s)
    pa, pb = _tc_dense2(z1, cnt2, b1r, Wnm, bnmr, w2a, w2b, w2c1, w2c2)
    pg = _sc_gather_pairsum(ei0g_w, ei1g_w, pa, pb, F2, EPAD)
    y = _tc_combine_y(pg[:E], efw, cnt2)
    z2 = _sc_scatter2(y, ei0g_w, ei1s_w, z128)
    return _tc_final(z2, y, cnt2, b2r)
